# Initial kernel scaffold; baseline (speedup 1.0000x reference)
import jax
import jax.numpy as jnp
from jax.experimental import pallas as pl


def kernel(donors_idx, mask_fit_X_col):
    raise NotImplementedError("write your pallas kernel here")


# speedup vs baseline: 1.5746x; 1.5746x over previous
import jax
import jax.numpy as jnp
from jax import lax
from jax.experimental import pallas as pl
from jax.experimental.pallas import tpu as pltpu
from jax.experimental.pallas import tpu_sc as plsc

_ROWS = 16384
_COLS = 100
_N = _ROWS * _COLS
_V = 1000000
_NW = 32
_PER_W = _N // _NW
_CHUNK = 6400
_W_BITS = 15
_WORDS = 1 << _W_BITS
_NBITS = 31
_PAD_V = _NBITS * _WORDS + _WORDS
_WPT = _WORDS // 16


def _sc_body(table_hbm, idx_hbm, out_hbm,
             colbuf, chunk, shared_packed, packed, idxbuf, outbuf, sem):
    c = lax.axis_index("c")
    s = lax.axis_index("s")

    wbase = s * _WPT
    copies = [
        pltpu.async_copy(
            table_hbm.at[pl.ds(b * _WORDS + wbase, _WPT)],
            colbuf.at[pl.ds(b * _WPT, _WPT)], sem)
        for b in range(_NBITS)
    ]
    for cp in copies:
        cp.wait()

    def pack_body(g, o):
        acc = jnp.zeros((16,), jnp.int32)
        for b in range(_NBITS):
            v = colbuf[pl.ds(jnp.int32(b * _WPT) + o, 16)]
            acc = acc | jnp.where(v != 0.0, jnp.int32(1 << b), jnp.int32(0))
        chunk[pl.ds(o, 16)] = acc
        return o + jnp.int32(16)

    lax.fori_loop(0, _WPT // 16, pack_body, jnp.int32(0))

    pltpu.sync_copy(chunk, shared_packed.at[pl.ds(wbase, _WPT)])
    plsc.subcore_barrier()
    pltpu.sync_copy(shared_packed, packed)

    wid = c * jnp.int32(16) + s
    base = wid * jnp.int32(_PER_W)

    def lookup_body(i, o):
        ivec = idxbuf[pl.ds(o, 16)]
        w = ivec & jnp.int32(_WORDS - 1)
        b = lax.shift_right_logical(ivec, jnp.int32(_W_BITS))
        word = plsc.load_gather(packed, [w])
        bit = lax.shift_right_logical(word, b) & jnp.int32(1)
        outbuf[pl.ds(o, 16)] = bit ^ jnp.int32(1)
        return o + jnp.int32(16)

    for cc in range(_PER_W // _CHUNK):
        off = base + jnp.int32(cc * _CHUNK)
        pltpu.sync_copy(idx_hbm.at[pl.ds(off, _CHUNK)], idxbuf)
        lax.fori_loop(0, _CHUNK // 16, lookup_body, jnp.int32(0))
        pltpu.sync_copy(outbuf, out_hbm.at[pl.ds(off, _CHUNK)])


def kernel(donors_idx, mask_fit_X_col):
    idx = donors_idx.reshape(-1).astype(jnp.int32)
    table = jnp.concatenate(
        [mask_fit_X_col.astype(jnp.float32),
         jnp.zeros((_PAD_V - _V,), jnp.float32)])

    mesh = plsc.VectorSubcoreMesh(core_axis_name="c", subcore_axis_name="s")
    out = pl.kernel(
        _sc_body,
        out_type=jax.ShapeDtypeStruct((_N,), jnp.int32),
        mesh=mesh,
        compiler_params=pltpu.CompilerParams(needs_layout_passes=False),
        scratch_types=[
            pltpu.VMEM((_NBITS * _WPT,), jnp.float32),
            pltpu.VMEM((_WPT,), jnp.int32),
            pltpu.VMEM_SHARED((_WORDS,), jnp.int32),
            pltpu.VMEM((_WORDS,), jnp.int32),
            pltpu.VMEM((_CHUNK,), jnp.int32),
            pltpu.VMEM((_CHUNK,), jnp.int32),
            pltpu.SemaphoreType.DMA,
        ],
    )(table, idx)
    return out.reshape(donors_idx.shape).astype(donors_idx.dtype)
